# baseline (speedup 1.0000x reference)
import functools

import jax
import jax.numpy as jnp
from jax import lax
from jax.experimental import pallas as pl
from jax.experimental.pallas import tpu as pltpu
from jax.experimental.pallas import tpu_sc as plsc

_INT_MIN = -(2 ** 31)
_L = 16
_NW = 32
_CAP = 256


def _vec(j):
    return pl.ds(j * _L, _L)


def _keys(v):
    u = lax.bitcast_convert_type(v, jnp.int32)
    return jnp.where(u >= 0, u, jnp.int32(_INT_MIN) - u)


def _key_to_f32(k):
    u = jnp.where(k >= 0, k, jnp.int32(_INT_MIN) - k)
    return lax.bitcast_convert_type(u, jnp.float32)


def _zero_hist(hist_ref):
    z = jnp.zeros((_L,), jnp.int32)

    def body(j, c):
        hist_ref[_vec(j)] = z
        return c

    lax.fori_loop(0, 256, body, 0)


def _scan_hist(hist_ref, tot_ref, kk):
    lanes = lax.iota(jnp.int32, _L)

    def tbody(j, c):
        def lbody(l, acc):
            return acc + hist_ref[pl.ds(l * 256 + j * _L, _L)]

        tot_ref[_vec(j)] = lax.fori_loop(0, _L, lbody, jnp.zeros((_L,), jnp.int32))
        return c

    lax.fori_loop(0, _L, tbody, 0)

    def sbody(i, carry):
        found, bstar, above, run = carry
        j = 15 - i
        t = tot_ref[_vec(j)]
        suf = lax.rev(plsc.cumsum(lax.rev(t, (0,))), (0,))
        hit = (run + suf) >= kk
        npc = jnp.sum(hit.astype(jnp.int32))
        lstar = npc - 1
        sel = lanes == lstar
        t_l = jnp.sum(jnp.where(sel, t, 0))
        s_l = jnp.sum(jnp.where(sel, suf, 0))
        got = (~found) & (npc > 0)
        bstar = jnp.where(got, j * _L + lstar, bstar)
        above = jnp.where(got, run + s_l - t_l, above)
        return (found | got, bstar, above, run + jnp.sum(t))

    _, bstar, above, _ = lax.fori_loop(
        0, _L, sbody,
        (jnp.bool_(False), jnp.int32(0), jnp.int32(0), jnp.int32(0)))
    return bstar, kk - above


def _radix_kth(src_ref, nvec, k, hist_ref, tot_ref, cand_ref):
    lanes = lax.iota(jnp.int32, _L)
    ones = jnp.ones((_L,), jnp.int32)
    slot_base = lanes * 256

    _zero_hist(hist_ref)

    def h1(j, c):
        kv = _keys(src_ref[_vec(j)])
        plsc.addupdate_scatter(hist_ref, [slot_base + ((kv >> 24) + 128)], ones)
        return c

    lax.fori_loop(0, nvec, h1, 0)
    b1, kk = _scan_hist(hist_ref, tot_ref, k)

    def c1(j, off):
        kv = _keys(src_ref[_vec(j)])
        m = ((kv >> 24) + 128) == b1
        pc = plsc.cumsum(m.astype(jnp.int32))
        plsc.store_scatter(cand_ref, [off + pc - 1], kv, mask=m)
        return off + jnp.sum(m.astype(jnp.int32))

    cnt = lax.fori_loop(0, nvec, c1, jnp.int32(0))

    tkey = (b1 - 128) << 24
    for shift in (16, 8, 0):
        _zero_hist(hist_ref)
        nv = (cnt + 15) >> 4

        def hl(j, c, cnt=cnt, shift=shift):
            kv = cand_ref[_vec(j)]
            valid = (j * _L + lanes) < cnt
            plsc.addupdate_scatter(
                hist_ref, [slot_base + ((kv >> shift) & 255)], ones,
                mask=valid)
            return c

        lax.fori_loop(0, nv, hl, 0)
        bs, kk = _scan_hist(hist_ref, tot_ref, kk)

        def cl(j, off, cnt=cnt, shift=shift, bs=bs):
            kv = cand_ref[_vec(j)]
            valid = ((j * _L + lanes) < cnt) & (((kv >> shift) & 255) == bs)
            pc = plsc.cumsum(valid.astype(jnp.int32))
            plsc.store_scatter(cand_ref, [off + pc - 1], kv, mask=valid)
            return off + jnp.sum(valid.astype(jnp.int32))

        cnt = lax.fori_loop(0, nv, cl, jnp.int32(0))
        tkey = tkey | (bs << shift)
    return tkey


def _binsearch_kth_nonneg(src_ref, nvec, k):
    def count_ge(f):
        def body(j, c):
            v = src_ref[_vec(j)]
            return c + jnp.sum((v >= f).astype(jnp.int32))

        return lax.fori_loop(0, nvec, body, jnp.int32(0))

    cur = jnp.int32(0)
    for b in range(30, -1, -1):
        test = cur + jnp.int32(2 ** b)
        fvec = _key_to_f32(jnp.broadcast_to(test, (_L,)))
        cnt = count_ge(fvec)
        cur = jnp.where(cnt >= k, test, cur)
    return cur


def _sc_select_body(pre_hbm, pre1_hbm, pre2_hbm, t0_hbm, t1_hbm, t2_hbm,
                    row_v, cand_v, hist_v, tot_v, pos_v, gat_v, c_v,
                    t0_v, t1_v, t2_v, sem):
    seq, sae_h = pre_hbm.shape
    nvec = sae_h // _L
    rpw = seq // _NW
    wid = lax.axis_index("s") * 2 + lax.axis_index("c")
    base = wid * rpw
    lanes = lax.iota(jnp.int32, _L)

    def row_body(i, carry):
        r = base + i
        pltpu.sync_copy(pre_hbm.at[r], row_v)

        tkey0 = _radix_kth(row_v, nvec, jnp.int32(150), hist_v, tot_v, cand_v)
        tau0 = _key_to_f32(jnp.broadcast_to(tkey0, (_L,)))

        zi = jnp.zeros((_L,), jnp.int32)

        def pz(j, c):
            pos_v[_vec(j)] = zi
            return c

        lax.fori_loop(0, _CAP // _L, pz, 0)
        rbase = r * sae_h

        def e1(j, off):
            v = row_v[_vec(j)]
            m = (v >= tau0) & (v != 0.0)
            pc = plsc.cumsum(m.astype(jnp.int32))
            idx = off + pc - 1
            plsc.store_scatter(pos_v, [idx], rbase + j * _L + lanes,
                               mask=m & (idx < _CAP))
            return off + jnp.sum(m.astype(jnp.int32))

        ncand = lax.fori_loop(0, nvec, e1, jnp.int32(0))
        ncand = jnp.minimum(ncand, jnp.int32(_CAP))

        for src, t_v in ((pre1_hbm, t1_v), (pre2_hbm, t2_v)):
            for g in range(_CAP // 128):
                pltpu.async_copy(
                    src.at[pos_v.at[pl.ds(g * 128, 128)]],
                    gat_v.at[pl.ds(g * 128, 128)], sem).wait()

            def cb(j, c):
                gv = gat_v[_vec(j)]
                valid = (j * _L + lanes) < ncand
                c_v[_vec(j)] = jnp.where(valid, jnp.maximum(gv, 0.0), 0.0)
                return c

            lax.fori_loop(0, _CAP // _L, cb, 0)
            tk = _binsearch_kth_nonneg(c_v, _CAP // _L, jnp.int32(75))
            tauc = _key_to_f32(jnp.broadcast_to(tk, (_L,)))
            plsc.store_scatter(t_v, [jnp.broadcast_to(i, (_L,))], tauc,
                               mask=lanes == 0)

        plsc.store_scatter(t0_v, [jnp.broadcast_to(i, (_L,))], tau0,
                           mask=lanes == 0)
        return carry

    lax.fori_loop(0, rpw, row_body, 0)
    pltpu.sync_copy(t0_v, t0_hbm.at[pl.ds(base, rpw)])
    pltpu.sync_copy(t1_v, t1_hbm.at[pl.ds(base, rpw)])
    pltpu.sync_copy(t2_v, t2_hbm.at[pl.ds(base, rpw)])


_DN_NT = (((1,), (1,)), ((), ()))


def _encode_body(x_ref, we_ref, be_ref, we1_ref, be1_ref, we2_ref, be2_ref,
                 p_ref, p1_ref, p2_ref):
    x = x_ref[...]
    p_ref[...] = jax.lax.dot_general(
        x, we_ref[...], _DN_NT, preferred_element_type=jnp.float32) + be_ref[...]
    p1_ref[...] = jax.lax.dot_general(
        x, we1_ref[...], _DN_NT, preferred_element_type=jnp.float32) + be1_ref[...]
    p2_ref[...] = jax.lax.dot_general(
        x, we2_ref[...], _DN_NT, preferred_element_type=jnp.float32) + be2_ref[...]


def _apply_decode_body(p_ref, p1_ref, p2_ref, t0_ref, t1_ref, t2_ref,
                       wd_ref, wd1_ref, wd2_ref, bsum_ref, out_ref):
    h = pl.program_id(0)
    pre = p_ref[...]
    t0 = t0_ref[...]
    mask = pre >= t0
    sae = jnp.where(mask, pre, 0.0).astype(jnp.bfloat16)
    maskc = mask & (pre != 0.0)
    m1 = jnp.where(maskc, p1_ref[...], 0.0)
    f1 = jnp.where(m1 >= t1_ref[...], m1, 0.0).astype(jnp.bfloat16)
    m2 = jnp.where(maskc, p2_ref[...], 0.0)
    f2 = jnp.where(m2 >= t2_ref[...], m2, 0.0).astype(jnp.bfloat16)

    part = jax.lax.dot_general(
        sae, wd_ref[...], _DN_NT, preferred_element_type=jnp.float32)
    part += jax.lax.dot_general(
        f1, wd1_ref[...], _DN_NT, preferred_element_type=jnp.float32)
    part += jax.lax.dot_general(
        f2, wd2_ref[...], _DN_NT, preferred_element_type=jnp.float32)

    @pl.when(h == 0)
    def _():
        out_ref[...] = part + bsum_ref[...]

    @pl.when(h != 0)
    def _():
        out_ref[...] += part


def kernel(llm_activations, We, be, Wd, bd, We1, be1, Wd1, bd1, We2, be2,
           Wd2, bd2):
    b, seq, llm_h = llm_activations.shape
    sae_h = We.shape[0]

    x = llm_activations.reshape(seq, llm_h)
    h_tile = min(512, sae_h)
    grid_h = sae_h // h_tile

    be_r = be.reshape(1, sae_h)
    be1_r = be1.reshape(1, sae_h)
    be2_r = be2.reshape(1, sae_h)
    bsum = (bd + bd1 + bd2).reshape(1, llm_h)

    f32 = jnp.float32
    pre_shape = jax.ShapeDtypeStruct((seq, sae_h), f32)

    w_spec = pl.BlockSpec((h_tile, llm_h), lambda h: (h, 0))
    bias_spec = pl.BlockSpec((1, h_tile), lambda h: (0, h))
    penc_spec = pl.BlockSpec((seq, h_tile), lambda h: (0, h))
    pre, pre1, pre2 = pl.pallas_call(
        _encode_body,
        grid=(grid_h,),
        in_specs=[
            pl.BlockSpec((seq, llm_h), lambda h: (0, 0)),
            w_spec, bias_spec, w_spec, bias_spec, w_spec, bias_spec,
        ],
        out_specs=[penc_spec, penc_spec, penc_spec],
        out_shape=[pre_shape, pre_shape, pre_shape],
    )(x, We, be_r, We1, be1_r, We2, be2_r)

    mesh = plsc.VectorSubcoreMesh(core_axis_name="c", subcore_axis_name="s")
    tau_t = jax.ShapeDtypeStruct((seq,), f32)
    rpw = seq // _NW
    sc_select = functools.partial(
        pl.kernel, mesh=mesh,
        compiler_params=pltpu.CompilerParams(needs_layout_passes=False),
        out_type=[tau_t, tau_t, tau_t],
        scratch_types=[
            pltpu.VMEM((sae_h,), f32),
            pltpu.VMEM((sae_h,), jnp.int32),
            pltpu.VMEM((4096,), jnp.int32),
            pltpu.VMEM((256,), jnp.int32),
            pltpu.VMEM((_CAP,), jnp.int32),
            pltpu.VMEM((_CAP,), f32),
            pltpu.VMEM((_CAP,), f32),
            pltpu.VMEM((rpw,), f32),
            pltpu.VMEM((rpw,), f32),
            pltpu.VMEM((rpw,), f32),
            pltpu.SemaphoreType.DMA,
        ],
    )(_sc_select_body)
    t0, t1, t2 = sc_select(pre, pre1.reshape(-1), pre2.reshape(-1))

    act_spec = pl.BlockSpec((seq, h_tile), lambda h: (0, h))
    wd_spec = pl.BlockSpec((llm_h, h_tile), lambda h: (0, h))
    tau_spec = pl.BlockSpec((seq, 1), lambda h: (0, 0))
    out = pl.pallas_call(
        _apply_decode_body,
        grid=(grid_h,),
        in_specs=[
            act_spec, act_spec, act_spec,
            tau_spec, tau_spec, tau_spec,
            wd_spec, wd_spec, wd_spec,
            pl.BlockSpec((1, llm_h), lambda h: (0, 0)),
        ],
        out_specs=pl.BlockSpec((seq, llm_h), lambda h: (0, 0)),
        out_shape=jax.ShapeDtypeStruct((seq, llm_h), f32),
    )(pre, pre1, pre2,
      t0.reshape(seq, 1), t1.reshape(seq, 1), t2.reshape(seq, 1),
      Wd.astype(jnp.bfloat16), Wd1.astype(jnp.bfloat16),
      Wd2.astype(jnp.bfloat16), bsum)

    return out.reshape(b, seq, llm_h)


# speedup vs baseline: 2.3129x; 2.3129x over previous
import functools

import jax
import jax.numpy as jnp
from jax import lax
from jax.experimental import pallas as pl
from jax.experimental.pallas import tpu as pltpu
from jax.experimental.pallas import tpu_sc as plsc

_INT_MIN = -(2 ** 31)
_L = 16
_NW = 32
_CHCAP = 256
_CAP = 256


def _vec(j):
    return pl.ds(j * _L, _L)


def _key_to_f32(k):
    u = jnp.where(k >= 0, k, jnp.int32(_INT_MIN) - k)
    return lax.bitcast_convert_type(u, jnp.float32)


def _binsearch_kth_nonneg(src_ref, nvec, k):
    cur = jnp.int32(0)
    for b in range(30, -1, -1):
        test = cur + jnp.int32(2 ** b)
        fvec = _key_to_f32(jnp.broadcast_to(test, (_L,)))
        cnt = jnp.int32(0)
        for j in range(nvec):
            cnt = cnt + jnp.sum((src_ref[_vec(j)] >= fvec).astype(jnp.int32))
        cur = jnp.where(cnt >= k, test, cur)
    return cur


def _sc_select_body(pre_hbm, gmax_hbm, tau0_hbm, pre1_hbm, pre2_hbm,
                    t1_hbm, t2_hbm,
                    row_v, gm_v, chid_v, pos_v, gat_v, c_v,
                    tau0_v, t1_v, t2_v, sem, rsem):
    seq, sae_h = pre_hbm.shape
    nchunk = sae_h // _L
    ncv = nchunk // _L
    rpw = seq // _NW
    wid = lax.axis_index("s") * 2 + lax.axis_index("c")
    base = wid * rpw
    lanes = lax.iota(jnp.int32, _L)
    zi = jnp.zeros((_L,), jnp.int32)

    pltpu.sync_copy(tau0_hbm.at[pl.ds(base, rpw)], tau0_v)

    def init(j, c):
        chid_v[_vec(j)] = zi
        pos_v[_vec(j)] = zi
        return c

    lax.fori_loop(0, _CHCAP // _L, init, 0)

    def row_body(i, carry):
        r = base + i
        row_cp = pltpu.async_copy(pre_hbm.at[r], row_v, rsem)
        pltpu.sync_copy(gmax_hbm.at[r], gm_v)
        tv = tau0_v[pl.ds((i >> 4) << 4, _L)]
        tau0 = jnp.broadcast_to(
            jnp.sum(jnp.where(lanes == (i & 15), tv, 0.0)), (_L,))

        def ce(j, off):
            m = gm_v[_vec(j)] >= tau0
            pc = plsc.cumsum(m.astype(jnp.int32))
            idx = off + pc - 1
            plsc.store_scatter(chid_v, [idx], j * _L + lanes,
                               mask=m & (idx < _CHCAP))
            return off + jnp.sum(m.astype(jnp.int32))

        nch = lax.fori_loop(0, ncv, ce, jnp.int32(0))
        nch = jnp.minimum(nch, jnp.int32(_CHCAP))
        row_cp.wait()

        rbase = r * sae_h

        def e1(j, off):
            cvec = chid_v[pl.ds((j >> 4) << 4, _L)]
            cid = jnp.sum(jnp.where(lanes == (j & 15), cvec, 0))
            mpos = cid + nchunk * lanes
            v = plsc.load_gather(row_v, [mpos])
            m = (v >= tau0) & (v != 0.0)
            pc = plsc.cumsum(m.astype(jnp.int32))
            idx = off + pc - 1
            plsc.store_scatter(pos_v, [idx], rbase + mpos,
                               mask=m & (idx < _CAP))
            return off + jnp.sum(m.astype(jnp.int32))

        ncand = lax.fori_loop(0, nch, e1, jnp.int32(0))
        ncand = jnp.minimum(ncand, jnp.int32(_CAP))

        for src, t_v in ((pre1_hbm, t1_v), (pre2_hbm, t2_v)):
            for g in range(_CAP // 128):
                pltpu.async_copy(
                    src.at[pos_v.at[pl.ds(g * 128, 128)]],
                    gat_v.at[pl.ds(g * 128, 128)], sem).wait()

            def cb(j, c):
                gv = gat_v[_vec(j)]
                valid = (j * _L + lanes) < ncand
                c_v[_vec(j)] = jnp.where(valid, jnp.maximum(gv, 0.0), 0.0)
                return c

            lax.fori_loop(0, _CAP // _L, cb, 0)
            tk = _binsearch_kth_nonneg(c_v, _CAP // _L, jnp.int32(75))
            tauc = _key_to_f32(jnp.broadcast_to(tk, (_L,)))
            plsc.store_scatter(t_v, [jnp.broadcast_to(i, (_L,))], tauc,
                               mask=lanes == 0)

        return carry

    lax.fori_loop(0, rpw, row_body, 0)
    pltpu.sync_copy(t1_v, t1_hbm.at[pl.ds(base, rpw)])
    pltpu.sync_copy(t2_v, t2_hbm.at[pl.ds(base, rpw)])


_DN_NT = (((1,), (1,)), ((), ()))


def _encode_body(x_ref, we_ref, be_ref, we1_ref, be1_ref, we2_ref, be2_ref,
                 p_ref, p1_ref, p2_ref):
    x = x_ref[...]
    p_ref[...] = jax.lax.dot_general(
        x, we_ref[...], _DN_NT, preferred_element_type=jnp.float32) + be_ref[...]
    p1_ref[...] = jax.lax.dot_general(
        x, we1_ref[...], _DN_NT, preferred_element_type=jnp.float32) + be1_ref[...]
    p2_ref[...] = jax.lax.dot_general(
        x, we2_ref[...], _DN_NT, preferred_element_type=jnp.float32) + be2_ref[...]


def _parent_tau_body(p_ref, gmax_ref, tau0_ref):
    pre = p_ref[...]
    rows, hh = pre.shape
    nchunk = hh // _L
    v = pre[:, :nchunk]
    for i in range(1, _L):
        v = jnp.maximum(v, pre[:, i * nchunk:(i + 1) * nchunk])
    gmax_ref[...] = v
    kf = jnp.float32(150)
    cnt0 = jnp.sum((pre >= 0.0).astype(jnp.float32), axis=1, keepdims=True)
    cur = jnp.where(cnt0 >= kf, jnp.int32(0), jnp.int32(_INT_MIN))
    cur = jnp.broadcast_to(cur, (rows, 1)).astype(jnp.int32)
    for b in range(30, -1, -1):
        test = cur + jnp.int32(2 ** b)
        u = jnp.where(test >= 0, test, jnp.int32(_INT_MIN) - test)
        f = jax.lax.bitcast_convert_type(u, jnp.float32)
        cnt = jnp.sum((pre >= f).astype(jnp.float32), axis=1, keepdims=True)
        cur = jnp.where(cnt >= kf, test, cur)
    u = jnp.where(cur >= 0, cur, jnp.int32(_INT_MIN) - cur)
    tau0_ref[...] = jax.lax.bitcast_convert_type(u, jnp.float32)


def _apply_decode_body(p_ref, p1_ref, p2_ref, t0_ref, t1_ref, t2_ref,
                       wd_ref, wd1_ref, wd2_ref, bsum_ref, out_ref):
    h = pl.program_id(0)
    pre = p_ref[...]
    t0 = t0_ref[...]
    mask = pre >= t0
    sae = jnp.where(mask, pre, 0.0).astype(jnp.bfloat16)
    maskc = mask & (pre != 0.0)
    m1 = jnp.where(maskc, p1_ref[...], 0.0)
    f1 = jnp.where(m1 >= t1_ref[...], m1, 0.0).astype(jnp.bfloat16)
    m2 = jnp.where(maskc, p2_ref[...], 0.0)
    f2 = jnp.where(m2 >= t2_ref[...], m2, 0.0).astype(jnp.bfloat16)

    part = jax.lax.dot_general(
        sae, wd_ref[...], _DN_NT, preferred_element_type=jnp.float32)
    part += jax.lax.dot_general(
        f1, wd1_ref[...], _DN_NT, preferred_element_type=jnp.float32)
    part += jax.lax.dot_general(
        f2, wd2_ref[...], _DN_NT, preferred_element_type=jnp.float32)

    @pl.when(h == 0)
    def _():
        out_ref[...] = part + bsum_ref[...]

    @pl.when(h != 0)
    def _():
        out_ref[...] += part


def kernel(llm_activations, We, be, Wd, bd, We1, be1, Wd1, bd1, We2, be2,
           Wd2, bd2):
    b, seq, llm_h = llm_activations.shape
    sae_h = We.shape[0]
    nchunk = sae_h // _L

    x = llm_activations.reshape(seq, llm_h)
    h_tile = min(512, sae_h)
    grid_h = sae_h // h_tile

    be_r = be.reshape(1, sae_h)
    be1_r = be1.reshape(1, sae_h)
    be2_r = be2.reshape(1, sae_h)
    bsum = (bd + bd1 + bd2).reshape(1, llm_h)

    f32 = jnp.float32
    pre_shape = jax.ShapeDtypeStruct((seq, sae_h), f32)

    w_spec = pl.BlockSpec((h_tile, llm_h), lambda h: (h, 0))
    bias_spec = pl.BlockSpec((1, h_tile), lambda h: (0, h))
    penc_spec = pl.BlockSpec((seq, h_tile), lambda h: (0, h))
    pre, pre1, pre2 = pl.pallas_call(
        _encode_body,
        grid=(grid_h,),
        in_specs=[
            pl.BlockSpec((seq, llm_h), lambda h: (0, 0)),
            w_spec, bias_spec, w_spec, bias_spec, w_spec, bias_spec,
        ],
        out_specs=[penc_spec, penc_spec, penc_spec],
        out_shape=[pre_shape, pre_shape, pre_shape],
    )(x, We, be_r, We1, be1_r, We2, be2_r)

    r_blk = min(64, seq)
    gmax, tau0 = pl.pallas_call(
        _parent_tau_body,
        grid=(seq // r_blk,),
        in_specs=[pl.BlockSpec((r_blk, sae_h), lambda t: (t, 0))],
        out_specs=[pl.BlockSpec((r_blk, nchunk), lambda t: (t, 0)),
                   pl.BlockSpec((r_blk, 1), lambda t: (t, 0))],
        out_shape=[jax.ShapeDtypeStruct((seq, nchunk), f32),
                   jax.ShapeDtypeStruct((seq, 1), f32)],
    )(pre)

    mesh = plsc.VectorSubcoreMesh(core_axis_name="c", subcore_axis_name="s")
    tau_t = jax.ShapeDtypeStruct((seq,), f32)
    rpw = seq // _NW
    sc_select = functools.partial(
        pl.kernel, mesh=mesh,
        compiler_params=pltpu.CompilerParams(needs_layout_passes=False),
        out_type=[tau_t, tau_t],
        scratch_types=[
            pltpu.VMEM((sae_h,), f32),
            pltpu.VMEM((nchunk,), f32),
            pltpu.VMEM((_CHCAP,), jnp.int32),
            pltpu.VMEM((_CAP,), jnp.int32),
            pltpu.VMEM((_CAP,), f32),
            pltpu.VMEM((_CAP,), f32),
            pltpu.VMEM((rpw,), f32),
            pltpu.VMEM((rpw,), f32),
            pltpu.VMEM((rpw,), f32),
            pltpu.SemaphoreType.DMA,
            pltpu.SemaphoreType.DMA,
        ],
    )(_sc_select_body)
    t1, t2 = sc_select(pre, gmax, tau0.reshape(-1),
                       pre1.reshape(-1), pre2.reshape(-1))

    act_spec = pl.BlockSpec((seq, h_tile), lambda h: (0, h))
    wd_spec = pl.BlockSpec((llm_h, h_tile), lambda h: (0, h))
    tau_spec = pl.BlockSpec((seq, 1), lambda h: (0, 0))
    out = pl.pallas_call(
        _apply_decode_body,
        grid=(grid_h,),
        in_specs=[
            act_spec, act_spec, act_spec,
            tau_spec, tau_spec, tau_spec,
            wd_spec, wd_spec, wd_spec,
            pl.BlockSpec((1, llm_h), lambda h: (0, 0)),
        ],
        out_specs=pl.BlockSpec((seq, llm_h), lambda h: (0, 0)),
        out_shape=jax.ShapeDtypeStruct((seq, llm_h), f32),
    )(pre, pre1, pre2,
      tau0, t1.reshape(seq, 1), t2.reshape(seq, 1),
      Wd.astype(jnp.bfloat16), Wd1.astype(jnp.bfloat16),
      Wd2.astype(jnp.bfloat16), bsum)

    return out.reshape(b, seq, llm_h)
